# baseline (speedup 1.0000x reference)
import functools

import jax
import jax.numpy as jnp
from jax import lax
from jax.experimental import pallas as pl
from jax.experimental.pallas import tpu as pltpu, tpu_sc as plsc

SCALE = 0.1
NC = 2
NS = 16
LANES = 16
BLK = 128
KC = 4
KHALF = 2


def _splat(x):
    return jnp.broadcast_to(jnp.asarray(x, jnp.int32), (LANES,))


def _sc_body(n_nodes, k_edges, d_out, t_hbm, tt_hbm, e_hbm, o_hbm,
             eblk, xit, idxc, xrows, stage, gsem, osem):
    nw = NC * NS
    n_blocks = (n_nodes + BLK - 1) // BLK
    n_items = n_blocks * KHALF
    kh = k_edges // KHALF
    n_chunks = kh // KC
    ng = BLK // LANES
    wid = lax.axis_index("s") * NC + lax.axis_index("c")
    count = (n_items - 1 - wid) // nw + 1

    iota = lax.iota(jnp.int32, LANES)

    def start_gathers(buf):
        for kk in range(KC):
            pltpu.async_copy(t_hbm.at[idxc.at[buf, kk]], xrows.at[buf, kk],
                             gsem[buf])

    def wait_gathers(buf):
        for kk in range(KC):
            pltpu.make_async_copy(t_hbm.at[idxc.at[0, 0]],
                                  xrows.at[buf, kk], gsem[buf]).wait()

    def build_idxc(k_base, c, buf):
        def g_body(g, carry):
            lane_g = iota + LANES * g
            for kk in range(KC):
                col = k_base + c * KC + kk
                v = plsc.load_gather(eblk, [lane_g, _splat(col)])
                idxc[buf, kk, pl.ds(LANES * g, LANES)] = v * LANES + iota
            return carry

        lax.fori_loop(0, ng, g_body, 0)

    def compute(c, buf):
        def g_body(g, carry):
            lane_g = iota + LANES * g
            off = LANES * g
            ci = xit[3 * 4, pl.ds(off, LANES)]
            smi = jnp.where(ci > 0.0, jnp.float32(SCALE), jnp.float32(0.0))
            col12 = (_splat(12) + iota) & 15
            s_kk = []
            for kk in range(KC):
                cj = plsc.load_gather(
                    xrows, [_splat(buf), _splat(kk), lane_g, col12])
                s_kk.append(jnp.where(cj > 0.0, smi, jnp.float32(0.0)))

            b_all = [[xit[3 * g1 + cc, pl.ds(off, LANES)] for cc in range(3)]
                     for g1 in range(4)]

            def g2_body(g2, carry2):
                r0 = g2 * 3
                for g1 in range(4):
                    b_cc = b_all[g1]
                    for cc in range(3):
                        rp = (_splat(r0 + cc) + iota) & 15
                        p = 12 * g1 + 3 * g2 + cc
                        for kk in range(KC):
                            a = plsc.load_gather(
                                xrows, [_splat(buf), _splat(kk), lane_g, rp])
                            stage[buf, kk, p, pl.ds(off, LANES)] = \
                                s_kk[kk] * (a - b_cc[cc])
                return carry2

            lax.fori_loop(0, 4, g2_body, 0)
            return carry

        lax.fori_loop(0, ng, g_body, 0)

    def item_body(i, carry):
        t = wid + nw * i
        blk = t // KHALF
        k_base = (t - blk * KHALF) * kh
        n0 = jnp.minimum(blk * BLK, n_nodes - BLK)
        pltpu.sync_copy(e_hbm.at[pl.ds(n0, BLK)], eblk)
        pltpu.sync_copy(tt_hbm.at[pl.ds(0, 16), pl.ds(n0, BLK)], xit)

        build_idxc(k_base, 0, 0)
        start_gathers(0)

        for c0 in range(0, n_chunks, 2):
            for par in range(2):
                c = c0 + par
                if c + 1 < n_chunks:
                    build_idxc(k_base, c + 1, (c + 1) % 2)
                    start_gathers((c + 1) % 2)
                wait_gathers(c % 2)
                if c >= 2:
                    pltpu.make_async_copy(
                        stage.at[c % 2],
                        o_hbm.at[pl.ds(0, KC), pl.ds(0, d_out),
                                 pl.ds(0, BLK)],
                        osem[c % 2]).wait()
                compute(c, c % 2)
                pltpu.async_copy(
                    stage.at[c % 2],
                    o_hbm.at[pl.ds(k_base + c * KC, KC), pl.ds(0, d_out),
                             pl.ds(n0, BLK)],
                    osem[c % 2])
        for buf in range(2):
            pltpu.make_async_copy(
                stage.at[buf],
                o_hbm.at[pl.ds(0, KC), pl.ds(0, d_out), pl.ds(0, BLK)],
                osem[buf]).wait()
        return carry

    lax.fori_loop(0, count, item_body, 0)


def _build_sc_call(n_nodes, k_edges, d_out):
    mesh = plsc.VectorSubcoreMesh(core_axis_name="c", subcore_axis_name="s")
    body = functools.partial(_sc_body, n_nodes, k_edges, d_out)
    return pl.kernel(
        body,
        out_type=jax.ShapeDtypeStruct((k_edges, d_out, n_nodes), jnp.float32),
        mesh=mesh,
        scratch_types=[
            pltpu.VMEM((BLK, k_edges + 1), jnp.int32),
            pltpu.VMEM((16, BLK), jnp.float32),
            pltpu.VMEM((2, KC, BLK), jnp.int32),
            pltpu.VMEM((2, KC, BLK, 16), jnp.float32),
            pltpu.VMEM((2, KC, d_out, BLK), jnp.float32),
            [pltpu.SemaphoreType.DMA, pltpu.SemaphoreType.DMA],
            [pltpu.SemaphoreType.DMA, pltpu.SemaphoreType.DMA],
        ],
        compiler_params=pltpu.CompilerParams(use_tc_tiling_on_sc=False,
                                             needs_layout_passes=False),
    )


def kernel(X, edge_idx, C):
    B, N, K = edge_idx.shape
    G = X.shape[2]
    d_out = 3 * G * G
    x2 = X.reshape(N, 3 * G)
    cf = C.reshape(N, 1).astype(jnp.float32)
    table = jnp.concatenate(
        [x2, cf, jnp.zeros((N, 16 - 3 * G - 1), jnp.float32)], axis=1)
    table_t = table.T
    table16 = jnp.stack([jnp.roll(table, r, axis=1) for r in range(16)],
                        axis=1).reshape(N * 16, 16)
    edges = jnp.pad(edge_idx.reshape(N, K).astype(jnp.int32),
                    ((0, 0), (0, 1)))
    call = _build_sc_call(N, K, d_out)
    out = call(table16, table_t, edges)
    return out.transpose(2, 0, 1).reshape(B, N, K, d_out)


# speedup vs baseline: 1.9672x; 1.9672x over previous
import functools

import jax
import jax.numpy as jnp
from jax import lax
from jax.experimental import pallas as pl
from jax.experimental.pallas import tpu as pltpu, tpu_sc as plsc

SCALE = 0.1
NC = 2
NS = 16
LANES = 16
BLK = 128
KC = 4
KHALF = 2


def _splat(x):
    return jnp.broadcast_to(jnp.asarray(x, jnp.int32), (LANES,))


def _sc_body(n_nodes, k_edges, d_out, t_hbm, tt_hbm, e_hbm, o_hbm,
             eblk, xit, idxc, xrows, stage, gsem, osem):
    nw = NC * NS
    n_blocks = (n_nodes + BLK - 1) // BLK
    n_items = n_blocks * KHALF
    kh = k_edges // KHALF
    n_chunks = kh // KC
    ng = BLK // LANES
    wid = lax.axis_index("s") * NC + lax.axis_index("c")
    count = (n_items - 1 - wid) // nw + 1

    iota = lax.iota(jnp.int32, LANES)

    def start_gathers(buf):
        for kk in range(KC):
            pltpu.async_copy(t_hbm.at[idxc.at[buf, kk]], xrows.at[buf, kk],
                             gsem[buf])

    def wait_gathers(buf):
        for kk in range(KC):
            pltpu.make_async_copy(t_hbm.at[idxc.at[0, 0]],
                                  xrows.at[buf, kk], gsem[buf]).wait()

    def build_idxc(k_base, c, buf):
        def g_body(g, carry):
            lane_g = iota + LANES * g
            for kk in range(KC):
                col = k_base + c * KC + kk
                v = plsc.load_gather(eblk, [lane_g, _splat(col)])
                idxc[buf, kk, pl.ds(LANES * g, LANES)] = v * LANES + iota
            return carry

        lax.fori_loop(0, ng, g_body, 0)

    def compute(c, buf):
        def g_body(g, carry):
            lane_g = iota + LANES * g
            off = LANES * g
            ci = xit[3 * 4, pl.ds(off, LANES)]
            smi = jnp.where(ci > 0.0, jnp.float32(SCALE), jnp.float32(0.0))
            col12 = (_splat(12) + iota) & 15
            s_kk = []
            for kk in range(KC):
                cj = plsc.load_gather(
                    xrows, [_splat(buf), _splat(kk), lane_g, col12])
                s_kk.append(jnp.where(cj > 0.0, smi, jnp.float32(0.0)))

            b_all = [[xit[3 * g1 + cc, pl.ds(off, LANES)] for cc in range(3)]
                     for g1 in range(4)]

            def g2_body(g2, carry2):
                r0 = g2 * 3
                a_kc = [[plsc.load_gather(
                    xrows, [_splat(buf), _splat(kk), lane_g,
                            (_splat(r0 + cc) + iota) & 15])
                    for cc in range(3)] for kk in range(KC)]
                for g1 in range(4):
                    b_cc = b_all[g1]
                    for cc in range(3):
                        p = 12 * g1 + 3 * g2 + cc
                        for kk in range(KC):
                            stage[buf, kk, p, pl.ds(off, LANES)] = \
                                s_kk[kk] * (a_kc[kk][cc] - b_cc[cc])
                return carry2

            lax.fori_loop(0, 4, g2_body, 0)
            return carry

        lax.fori_loop(0, ng, g_body, 0)

    def item_body(i, carry):
        t = wid + nw * i
        blk = t // KHALF
        k_base = (t - blk * KHALF) * kh
        n0 = jnp.minimum(blk * BLK, n_nodes - BLK)
        pltpu.sync_copy(e_hbm.at[pl.ds(n0, BLK)], eblk)
        pltpu.sync_copy(tt_hbm.at[pl.ds(0, 16), pl.ds(n0, BLK)], xit)

        build_idxc(k_base, 0, 0)
        start_gathers(0)

        for c0 in range(0, n_chunks, 2):
            for par in range(2):
                c = c0 + par
                if c + 1 < n_chunks:
                    build_idxc(k_base, c + 1, (c + 1) % 2)
                    start_gathers((c + 1) % 2)
                wait_gathers(c % 2)
                if c >= 2:
                    pltpu.make_async_copy(
                        stage.at[c % 2],
                        o_hbm.at[pl.ds(0, KC), pl.ds(0, d_out),
                                 pl.ds(0, BLK)],
                        osem[c % 2]).wait()
                compute(c, c % 2)
                pltpu.async_copy(
                    stage.at[c % 2],
                    o_hbm.at[pl.ds(k_base + c * KC, KC), pl.ds(0, d_out),
                             pl.ds(n0, BLK)],
                    osem[c % 2])
        for buf in range(2):
            pltpu.make_async_copy(
                stage.at[buf],
                o_hbm.at[pl.ds(0, KC), pl.ds(0, d_out), pl.ds(0, BLK)],
                osem[buf]).wait()
        return carry

    lax.fori_loop(0, count, item_body, 0)


def _build_sc_call(n_nodes, k_edges, d_out):
    mesh = plsc.VectorSubcoreMesh(core_axis_name="c", subcore_axis_name="s")
    body = functools.partial(_sc_body, n_nodes, k_edges, d_out)
    return pl.kernel(
        body,
        out_type=jax.ShapeDtypeStruct((k_edges, d_out, n_nodes), jnp.float32),
        mesh=mesh,
        scratch_types=[
            pltpu.VMEM((BLK, k_edges + 1), jnp.int32),
            pltpu.VMEM((16, BLK), jnp.float32),
            pltpu.VMEM((2, KC, BLK), jnp.int32),
            pltpu.VMEM((2, KC, BLK, 16), jnp.float32),
            pltpu.VMEM((2, KC, d_out, BLK), jnp.float32),
            [pltpu.SemaphoreType.DMA, pltpu.SemaphoreType.DMA],
            [pltpu.SemaphoreType.DMA, pltpu.SemaphoreType.DMA],
        ],
        compiler_params=pltpu.CompilerParams(use_tc_tiling_on_sc=False,
                                             needs_layout_passes=False),
    )


def kernel(X, edge_idx, C):
    B, N, K = edge_idx.shape
    G = X.shape[2]
    d_out = 3 * G * G
    x2 = X.reshape(N, 3 * G)
    cf = C.reshape(N, 1).astype(jnp.float32)
    table = jnp.concatenate(
        [x2, cf, jnp.zeros((N, 16 - 3 * G - 1), jnp.float32)], axis=1)
    table_t = table.T
    table16 = jnp.stack([jnp.roll(table, r, axis=1) for r in range(16)],
                        axis=1).reshape(N * 16, 16)
    edges = jnp.pad(edge_idx.reshape(N, K).astype(jnp.int32),
                    ((0, 0), (0, 1)))
    call = _build_sc_call(N, K, d_out)
    out = call(table16, table_t, edges)
    return out.transpose(2, 0, 1).reshape(B, N, K, d_out)
